# baseline (speedup 1.0000x reference)
import numpy as np
import jax
import jax.numpy as jnp
from jax.experimental import pallas as pl
from jax.experimental.pallas import tpu as pltpu

_NCLS = 64
_T = 100
_TPAD = 128
_LOG_NC = float(np.log(_NCLS))


def _block_body(ts_ref, la_ref, l1ma_ref, pk_ref, u_ref,
                vp_ref, lnvt_ref, lv0_ref):
    f32 = jnp.float32
    log_eps = jnp.log(f32(1e-30))

    def lae(a, b):
        m = jnp.maximum(a, b)
        return m + jnp.log(jnp.exp(a - m) + jnp.exp(b - m))

    la = la_ref[...]
    l1ma = l1ma_ref[...]
    rest = l1ma - _LOG_NC
    on_col = lae(la, rest)
    off_col = lae(la + log_eps, rest)

    ts = ts_ref[...]
    iota_sub = jax.lax.broadcasted_iota(jnp.int32, (_TPAD, _NCLS), 0)
    mt = ts == iota_sub
    on_b = jnp.sum(jnp.where(mt, on_col, f32(0.0)), axis=0, keepdims=True)
    off_b = jnp.sum(jnp.where(mt, off_col, f32(0.0)), axis=0, keepdims=True)

    pk = pk_ref[...]
    vcls = pk[:, 0:1].astype(jnp.int32)
    bidx = pk[:, 1:2].astype(jnp.int32)
    iota64 = jax.lax.broadcasted_iota(jnp.int32, (1, _NCLS), 1)
    mb = bidx == iota64
    on_n = jnp.sum(jnp.where(mb, on_b, f32(0.0)), axis=1, keepdims=True)
    off_n = jnp.sum(jnp.where(mb, off_b, f32(0.0)), axis=1, keepdims=True)

    u = u_ref[...]
    g = -jnp.log(-jnp.log(u + f32(1e-30)) + f32(1e-30))
    mv = vcls == iota64
    val = g + jnp.where(mv, on_n, off_n)
    vmax = jnp.max(val, axis=1, keepdims=True)
    ms = val == vmax

    vp_ref[...] = jnp.where(ms, f32(1.0), f32(0.0))
    lnvt_ref[...] = jnp.where(ms, f32(0.0), log_eps)
    lv0_ref[...] = jnp.where(mv, f32(0.0), log_eps)


def kernel(v, time_step, batch, u, log_alphas_bar, log_1_min_alphas_bar):
    n = u.shape[0]
    rows = 8192
    grid = n // rows
    ts2 = time_step.reshape(1, _NCLS)
    la2 = jnp.pad(log_alphas_bar, (0, _TPAD - _T)).reshape(_TPAD, 1)
    l12 = jnp.pad(log_1_min_alphas_bar, (0, _TPAD - _T)).reshape(_TPAD, 1)
    pk = jnp.stack([v, batch], axis=1).astype(jnp.int8)

    grid_spec = pl.GridSpec(
        grid=(grid,),
        in_specs=[
            pl.BlockSpec((1, _NCLS), lambda i: (0, 0)),
            pl.BlockSpec((_TPAD, 1), lambda i: (0, 0)),
            pl.BlockSpec((_TPAD, 1), lambda i: (0, 0)),
            pl.BlockSpec((rows, 2), lambda i: (i, 0)),
            pl.BlockSpec((rows, _NCLS), lambda i: (i, 0)),
        ],
        out_specs=[pl.BlockSpec((rows, _NCLS), lambda i: (i, 0))] * 3,
    )
    vp, lnvt, lv0 = pl.pallas_call(
        _block_body,
        grid_spec=grid_spec,
        out_shape=[jax.ShapeDtypeStruct((n, _NCLS), jnp.float32)] * 3,
        compiler_params=pltpu.CompilerParams(
            dimension_semantics=("parallel",)),
    )(ts2, la2, l12, pk, u)
    return (vp, lnvt, lv0)


# speedup vs baseline: 1.0144x; 1.0144x over previous
import numpy as np
import jax
import jax.numpy as jnp
from jax.experimental import pallas as pl
from jax.experimental.pallas import tpu as pltpu

_NCLS = 64
_T = 100
_TPAD = 128
_LOG_NC = float(np.log(_NCLS))


def _block_body(ts_ref, la_ref, l1ma_ref, pk_ref, u_ref,
                vp_ref, lnvt_ref, lv0_ref):
    f32 = jnp.float32
    log_eps = jnp.log(f32(1e-30))

    def lae(a, b):
        m = jnp.maximum(a, b)
        return m + jnp.log(jnp.exp(a - m) + jnp.exp(b - m))

    la = la_ref[...]
    l1ma = l1ma_ref[...]
    rest = l1ma - _LOG_NC
    on_col = lae(la, rest)
    off_col = lae(la + log_eps, rest)

    ts = ts_ref[...]
    iota_sub = jax.lax.broadcasted_iota(jnp.int32, (_TPAD, _NCLS), 0)
    mt = ts == iota_sub
    on_b = jnp.sum(jnp.where(mt, on_col, f32(0.0)), axis=0, keepdims=True)
    off_b = jnp.sum(jnp.where(mt, off_col, f32(0.0)), axis=0, keepdims=True)

    pk = pk_ref[...].astype(jnp.int32)
    bidx = jax.lax.shift_right_logical(pk, 6)
    vcls = jax.lax.bitwise_and(pk, _NCLS - 1)
    iota64 = jax.lax.broadcasted_iota(jnp.int32, (1, _NCLS), 1)
    mb = bidx == iota64
    on_n = jnp.sum(jnp.where(mb, on_b, f32(0.0)), axis=1, keepdims=True)
    off_n = jnp.sum(jnp.where(mb, off_b, f32(0.0)), axis=1, keepdims=True)

    u = u_ref[...]
    g = -jnp.log(-jnp.log(u + f32(1e-30)) + f32(1e-30))
    mv = vcls == iota64
    val = g + jnp.where(mv, on_n, off_n)
    vmax = jnp.max(val, axis=1, keepdims=True)
    ms = val == vmax

    vp_ref[...] = jnp.where(ms, f32(1.0), f32(0.0))
    lnvt_ref[...] = jnp.where(ms, f32(0.0), log_eps)
    lv0_ref[...] = jnp.where(mv, f32(0.0), log_eps)


def kernel(v, time_step, batch, u, log_alphas_bar, log_1_min_alphas_bar):
    n = u.shape[0]
    rows = 8192
    grid = n // rows
    ts2 = time_step.reshape(1, _NCLS)
    la2 = jnp.pad(log_alphas_bar, (0, _TPAD - _T)).reshape(_TPAD, 1)
    l12 = jnp.pad(log_1_min_alphas_bar, (0, _TPAD - _T)).reshape(_TPAD, 1)
    pk = (batch * _NCLS + v).astype(jnp.int16).reshape(n, 1)

    grid_spec = pl.GridSpec(
        grid=(grid,),
        in_specs=[
            pl.BlockSpec((1, _NCLS), lambda i: (0, 0)),
            pl.BlockSpec((_TPAD, 1), lambda i: (0, 0)),
            pl.BlockSpec((_TPAD, 1), lambda i: (0, 0)),
            pl.BlockSpec((rows, 1), lambda i: (i, 0)),
            pl.BlockSpec((rows, _NCLS), lambda i: (i, 0)),
        ],
        out_specs=[pl.BlockSpec((rows, _NCLS), lambda i: (i, 0))] * 3,
    )
    vp, lnvt, lv0 = pl.pallas_call(
        _block_body,
        grid_spec=grid_spec,
        out_shape=[jax.ShapeDtypeStruct((n, _NCLS), jnp.float32)] * 3,
        compiler_params=pltpu.CompilerParams(
            dimension_semantics=("parallel",)),
    )(ts2, la2, l12, pk, u)
    return (vp, lnvt, lv0)


# speedup vs baseline: 1.0314x; 1.0167x over previous
import numpy as np
import jax
import jax.numpy as jnp
from jax.experimental import pallas as pl
from jax.experimental.pallas import tpu as pltpu

_NCLS = 64
_T = 100
_TPAD = 128
_LOG_NC = float(np.log(_NCLS))


def _block_body(ts_ref, la_ref, l1ma_ref, pk_ref, u_ref,
                vp_ref, lnvt_ref, lv0_ref):
    f32 = jnp.float32
    log_eps = jnp.log(f32(1e-30))

    def lae(a, b):
        m = jnp.maximum(a, b)
        return m + jnp.log(jnp.exp(a - m) + jnp.exp(b - m))

    la = la_ref[...]
    l1ma = l1ma_ref[...]
    rest = l1ma - _LOG_NC
    on_col = lae(la, rest)
    off_col = lae(la + log_eps, rest)

    ts = ts_ref[...]
    iota_sub = jax.lax.broadcasted_iota(jnp.int32, (_TPAD, _NCLS), 0)
    mt = ts == iota_sub
    delta_b = jnp.sum(jnp.where(mt, on_col - off_col, f32(0.0)), axis=0,
                      keepdims=True)

    pk = pk_ref[...].astype(jnp.int32)
    bidx = jax.lax.shift_right_logical(pk, 6)
    vcls = jax.lax.bitwise_and(pk, _NCLS - 1)
    iota64 = jax.lax.broadcasted_iota(jnp.int32, (1, _NCLS), 1)
    mb = bidx == iota64
    delta_n = jnp.sum(jnp.where(mb, delta_b, f32(0.0)), axis=1, keepdims=True)

    u = u_ref[...]
    g = -jnp.log(-jnp.log(u + f32(1e-30)) + f32(1e-30))
    mv = vcls == iota64
    val = g + jnp.where(mv, delta_n, f32(0.0))
    vmax = jnp.max(val, axis=1, keepdims=True)
    ms = val == vmax

    vp_ref[...] = jnp.where(ms, f32(1.0), f32(0.0))
    lnvt_ref[...] = jnp.where(ms, f32(0.0), log_eps)
    lv0_ref[...] = jnp.where(mv, f32(0.0), log_eps)


def kernel(v, time_step, batch, u, log_alphas_bar, log_1_min_alphas_bar):
    n = u.shape[0]
    rows = 8192
    grid = n // rows
    ts2 = time_step.reshape(1, _NCLS)
    la2 = jnp.pad(log_alphas_bar, (0, _TPAD - _T)).reshape(_TPAD, 1)
    l12 = jnp.pad(log_1_min_alphas_bar, (0, _TPAD - _T)).reshape(_TPAD, 1)
    pk = (batch * _NCLS + v).astype(jnp.int16).reshape(n, 1)

    grid_spec = pl.GridSpec(
        grid=(grid,),
        in_specs=[
            pl.BlockSpec((1, _NCLS), lambda i: (0, 0)),
            pl.BlockSpec((_TPAD, 1), lambda i: (0, 0)),
            pl.BlockSpec((_TPAD, 1), lambda i: (0, 0)),
            pl.BlockSpec((rows, 1), lambda i: (i, 0)),
            pl.BlockSpec((rows, _NCLS), lambda i: (i, 0)),
        ],
        out_specs=[pl.BlockSpec((rows, _NCLS), lambda i: (i, 0))] * 3,
    )
    vp, lnvt, lv0 = pl.pallas_call(
        _block_body,
        grid_spec=grid_spec,
        out_shape=[jax.ShapeDtypeStruct((n, _NCLS), jnp.float32)] * 3,
        compiler_params=pltpu.CompilerParams(
            dimension_semantics=("parallel",)),
    )(ts2, la2, l12, pk, u)
    return (vp, lnvt, lv0)
